# baseline (speedup 1.0000x reference)
import functools

import jax
import jax.numpy as jnp
from jax import lax
from jax.experimental import pallas as pl
from jax.experimental.pallas import tpu as pltpu
from jax.experimental.pallas import tpu_sc as plsc

N = 16384
NUM_TYPES_ROWS = 100000
EMB_DIM = 64
CHG_DIM = 8
OUT_DIM = EMB_DIM + CHG_DIM

_info = plsc.get_sparse_core_info()
NC, NS = _info.num_cores, _info.num_subcores
NW = NC * NS
D_PER_W = EMB_DIM // NW

IDX_CHUNK_ROWS = 32
N_IDX_CHUNKS = 128 // IDX_CHUNK_ROWS

TC_DIM = OUT_DIM // 8
TR_DIM = N // 128

_mesh = plsc.VectorSubcoreMesh(core_axis_name="c", subcore_axis_name="s")


@functools.partial(
    pl.kernel,
    mesh=_mesh,
    out_type=jax.ShapeDtypeStruct((TC_DIM, TR_DIM, 1024), jnp.float32),
    scratch_types=[
        pltpu.VMEM((NUM_TYPES_ROWS,), jnp.float32),
        pltpu.VMEM((IDX_CHUNK_ROWS, 128), jnp.int32),
        pltpu.VMEM((IDX_CHUNK_ROWS, 128), jnp.int32),
        pltpu.VMEM((TR_DIM, 128), jnp.float32),
        pltpu.SemaphoreType.DMA,
        pltpu.SemaphoreType.DMA,
    ],
    compiler_params=pltpu.CompilerParams(use_tc_tiling_on_sc=False,
                                         needs_layout_passes=False),
)
def _emb_kernel(idx_hbm, chargeT_hbm, tableT_hbm, out_hbm,
                tb_v, idxc_a, idxc_b, gath_v, sem, sem2):
    wid = lax.axis_index("s") * NC + lax.axis_index("c")
    idxc = (idxc_a, idxc_b)

    for dd in range(D_PER_W):
        d = wid * D_PER_W + dd
        pend = pltpu.async_copy(
            idx_hbm.at[pl.ds(0, IDX_CHUNK_ROWS)], idxc[0], sem2)
        pltpu.sync_copy(tableT_hbm.at[d], tb_v)

        for p in range(N_IDX_CHUNKS):
            pend.wait()
            if p + 1 < N_IDX_CHUNKS:
                pend = pltpu.async_copy(
                    idx_hbm.at[pl.ds((p + 1) * IDX_CHUNK_ROWS,
                                     IDX_CHUNK_ROWS)],
                    idxc[(p + 1) % 2], sem2)
            buf = idxc[p % 2]

            def row_body(r, carry, p=p, buf=buf):
                for lb in range(8):
                    iv = buf[r, pl.ds(lb * 16, 16)]
                    v = plsc.load_gather(tb_v, [iv])
                    gath_v[p * IDX_CHUNK_ROWS + r, pl.ds(lb * 16, 16)] = v
                return carry

            lax.fori_loop(0, IDX_CHUNK_ROWS, row_body, 0, unroll=4)

        pltpu.sync_copy(
            gath_v,
            out_hbm.at[d // 8, pl.ds(0, TR_DIM), pl.ds((d % 8) * 128, 128)],
        )

    @pl.when(wid < CHG_DIM)
    def _():
        pltpu.sync_copy(
            chargeT_hbm.at[wid],
            out_hbm.at[EMB_DIM // 8, pl.ds(0, TR_DIM),
                       pl.ds(wid * 128, 128)],
        )


def kernel(atom_types, charge, pos, emb_table):
    idx = atom_types.astype(jnp.int32).reshape(128, 128)
    chargeT = charge.T.reshape(CHG_DIM, TR_DIM, 128)
    out4 = _emb_kernel(idx, chargeT, emb_table.T)
    out = (out4.reshape(TC_DIM, TR_DIM, 8, 128)
           .transpose(1, 3, 0, 2).reshape(N, OUT_DIM))
    return out.astype(pos.dtype)


# speedup vs baseline: 1.0061x; 1.0061x over previous
import functools

import jax
import jax.numpy as jnp
from jax import lax
from jax.experimental import pallas as pl
from jax.experimental.pallas import tpu as pltpu
from jax.experimental.pallas import tpu_sc as plsc

N = 16384
NUM_TYPES_ROWS = 100000
EMB_DIM = 64
CHG_DIM = 8
OUT_DIM = EMB_DIM + CHG_DIM

_info = plsc.get_sparse_core_info()
NC, NS = _info.num_cores, _info.num_subcores
NW = NC * NS
D_PER_W = EMB_DIM // NW

IDX_CHUNK_ROWS = 32
N_IDX_CHUNKS = 128 // IDX_CHUNK_ROWS

TC_DIM = OUT_DIM // 8
TR_DIM = N // 128

_mesh = plsc.VectorSubcoreMesh(core_axis_name="c", subcore_axis_name="s")


@functools.partial(
    pl.kernel,
    mesh=_mesh,
    out_type=jax.ShapeDtypeStruct((TC_DIM, TR_DIM, 1024), jnp.float32),
    scratch_types=[
        pltpu.VMEM((NUM_TYPES_ROWS,), jnp.float32),
        pltpu.VMEM((IDX_CHUNK_ROWS, 128), jnp.int32),
        pltpu.VMEM((IDX_CHUNK_ROWS, 128), jnp.int32),
        pltpu.VMEM((TR_DIM, 128), jnp.float32),
        pltpu.SemaphoreType.DMA,
        pltpu.SemaphoreType.DMA,
    ],
    compiler_params=pltpu.CompilerParams(use_tc_tiling_on_sc=False,
                                         needs_layout_passes=False),
)
def _emb_kernel(idx_hbm, chargeT_hbm, tableT_hbm, out_hbm,
                tb_v, idxc_a, idxc_b, gath_v, sem, sem2):
    wid = lax.axis_index("s") * NC + lax.axis_index("c")
    idxc = (idxc_a, idxc_b)

    for dd in range(D_PER_W):
        d = wid * D_PER_W + dd
        pend = pltpu.async_copy(
            idx_hbm.at[pl.ds(0, IDX_CHUNK_ROWS)], idxc[0], sem2)
        pltpu.sync_copy(tableT_hbm.at[d], tb_v)

        for p in range(N_IDX_CHUNKS):
            pend.wait()
            if p + 1 < N_IDX_CHUNKS:
                pend = pltpu.async_copy(
                    idx_hbm.at[pl.ds((p + 1) * IDX_CHUNK_ROWS,
                                     IDX_CHUNK_ROWS)],
                    idxc[(p + 1) % 2], sem2)
            buf = idxc[p % 2]

            def row_body(r, carry, p=p, buf=buf):
                for lb in range(8):
                    iv = buf[r, pl.ds(lb * 16, 16)]
                    v = plsc.load_gather(tb_v, [iv])
                    gath_v[p * IDX_CHUNK_ROWS + r, pl.ds(lb * 16, 16)] = v
                return carry

            lax.fori_loop(0, IDX_CHUNK_ROWS, row_body, 0, unroll=2)

        pltpu.sync_copy(
            gath_v,
            out_hbm.at[d // 8, pl.ds(0, TR_DIM), pl.ds((d % 8) * 128, 128)],
        )

    @pl.when(wid < CHG_DIM)
    def _():
        pltpu.sync_copy(
            chargeT_hbm.at[wid],
            out_hbm.at[EMB_DIM // 8, pl.ds(0, TR_DIM),
                       pl.ds(wid * 128, 128)],
        )


def kernel(atom_types, charge, pos, emb_table):
    idx = atom_types.astype(jnp.int32).reshape(128, 128)
    chargeT = charge.T.reshape(CHG_DIM, TR_DIM, 128)
    out4 = _emb_kernel(idx, chargeT, emb_table.T)
    out = (out4.reshape(TC_DIM, TR_DIM, 8, 128)
           .transpose(1, 3, 0, 2).reshape(N, OUT_DIM))
    return out.astype(pos.dtype)


# speedup vs baseline: 1.3972x; 1.3888x over previous
import functools

import jax
import jax.numpy as jnp
from jax import lax
from jax.experimental import pallas as pl
from jax.experimental.pallas import tpu as pltpu
from jax.experimental.pallas import tpu_sc as plsc

N = 16384
NUM_TYPES_ROWS = 100000
EMB_DIM = 64
CHG_DIM = 8
OUT_DIM = EMB_DIM + CHG_DIM

_info = plsc.get_sparse_core_info()
NC, NS = _info.num_cores, _info.num_subcores
NW = NC * NS
D_PER_W = EMB_DIM // NW

IDX_CHUNK_ROWS = 32
N_IDX_CHUNKS = 128 // IDX_CHUNK_ROWS

TC_DIM = OUT_DIM // 8
TR_DIM = N // 128

_mesh = plsc.VectorSubcoreMesh(core_axis_name="c", subcore_axis_name="s")


@functools.partial(
    pl.kernel,
    mesh=_mesh,
    out_type=jax.ShapeDtypeStruct((TC_DIM, TR_DIM, 1024), jnp.float32),
    scratch_types=[
        pltpu.VMEM((NUM_TYPES_ROWS,), jnp.float32),
        pltpu.VMEM((IDX_CHUNK_ROWS, 128), jnp.int32),
        pltpu.VMEM((IDX_CHUNK_ROWS, 128), jnp.int32),
        pltpu.VMEM((TR_DIM, 128), jnp.float32),
        pltpu.SemaphoreType.DMA,
        pltpu.SemaphoreType.DMA,
    ],
    compiler_params=pltpu.CompilerParams(needs_layout_passes=False),
)
def _emb_kernel(idx_hbm, chargeT_hbm, tableT_hbm, out_hbm,
                tb_v, idxc_a, idxc_b, gath_v, sem, sem2):
    wid = lax.axis_index("s") * NC + lax.axis_index("c")
    idxc = (idxc_a, idxc_b)

    for dd in range(D_PER_W):
        d = wid * D_PER_W + dd
        pend = pltpu.async_copy(
            idx_hbm.at[pl.ds(0, IDX_CHUNK_ROWS)], idxc[0], sem2)
        pltpu.sync_copy(tableT_hbm.at[d], tb_v)

        for p in range(N_IDX_CHUNKS):
            pend.wait()
            if p + 1 < N_IDX_CHUNKS:
                pend = pltpu.async_copy(
                    idx_hbm.at[pl.ds((p + 1) * IDX_CHUNK_ROWS,
                                     IDX_CHUNK_ROWS)],
                    idxc[(p + 1) % 2], sem2)
            buf = idxc[p % 2]

            def row_body(r, carry, p=p, buf=buf):
                for lb in range(8):
                    iv = buf[r, pl.ds(lb * 16, 16)]
                    v = plsc.load_gather(tb_v, [iv])
                    gath_v[p * IDX_CHUNK_ROWS + r, pl.ds(lb * 16, 16)] = v
                return carry

            lax.fori_loop(0, IDX_CHUNK_ROWS, row_body, 0, unroll=2)

        pltpu.sync_copy(
            gath_v,
            out_hbm.at[d // 8, pl.ds(0, TR_DIM), pl.ds((d % 8) * 128, 128)],
        )

    @pl.when(wid < CHG_DIM)
    def _():
        pltpu.sync_copy(
            chargeT_hbm.at[wid],
            out_hbm.at[EMB_DIM // 8, pl.ds(0, TR_DIM),
                       pl.ds(wid * 128, 128)],
        )


def kernel(atom_types, charge, pos, emb_table):
    idx = atom_types.astype(jnp.int32).reshape(128, 128)
    chargeT = charge.T.reshape(CHG_DIM, TR_DIM, 128)
    out4 = _emb_kernel(idx, chargeT, emb_table.T)
    out = (out4.reshape(TC_DIM, TR_DIM, 8, 128)
           .transpose(1, 3, 0, 2).reshape(N, OUT_DIM))
    return out.astype(pos.dtype)
